# baseline (speedup 1.0000x reference)
import jax
import jax.numpy as jnp
from jax.experimental import pallas as pl

ITEMS = 100000
D = 32
B = 8
BLK = 2000


def _blend_body(x_ref, e_ref, a_ref, o_ref):
    x = x_ref[...]
    e = e_ref[...]
    a = a_ref[...]
    o_ref[...] = e[None, :, :] + a[None, :, :] * (x - e[None, :, :])


def kernel(nodes_output, embedding_table, alpha):
    nodes = nodes_output.reshape(B, ITEMS, D)
    grid = (ITEMS // BLK,)
    return pl.pallas_call(
        _blend_body,
        grid=grid,
        in_specs=[
            pl.BlockSpec((B, BLK, D), lambda i: (0, i, 0)),
            pl.BlockSpec((BLK, D), lambda i: (i, 0)),
            pl.BlockSpec((BLK, 1), lambda i: (i, 0)),
        ],
        out_specs=pl.BlockSpec((B, BLK, D), lambda i: (0, i, 0)),
        out_shape=jax.ShapeDtypeStruct((B, ITEMS, D), jnp.float32),
    )(nodes, embedding_table, alpha)
